# baseline (speedup 1.0000x reference)
import jax
import jax.numpy as jnp
from jax import lax
from jax.experimental import pallas as pl
from jax.experimental.pallas import tpu as pltpu
from jax.experimental.pallas import tpu_sc as plsc

VOCAB = 100000
EMBED = 128
BATCH = 4096
NUM_CTX = 5

NUM_CORES = 2
NUM_SUBCORES = 16
NUM_WORKERS = NUM_CORES * NUM_SUBCORES
B_PER_W = BATCH // NUM_WORKERS
CHUNK = 64
NUM_CHUNKS = B_PER_W // CHUNK
LANES = 16
K_CHUNKS = EMBED // LANES


def _sc_kernel_body(tgt_idx_hbm, ctx_idx_hbm, w_tgt_hbm, w_ctx_hbm, out_hbm,
                    tgt_idx_v, ctx_idx_v, tgt_rows, ctx_rows, out_v, sems):
    wid = lax.axis_index("subcore") * NUM_CORES + lax.axis_index("core")
    base = wid * B_PER_W

    pltpu.sync_copy(tgt_idx_hbm.at[pl.ds(base, B_PER_W)], tgt_idx_v)
    pltpu.sync_copy(ctx_idx_hbm.at[:, pl.ds(base, B_PER_W)], ctx_idx_v)

    last_lane = lax.iota(jnp.int32, LANES) == (LANES - 1)

    HALF = CHUNK // 2

    def issue_gathers(g, par):
        cps = []
        for h in range(2):
            cps.append(pltpu.async_copy(
                w_tgt_hbm.at[tgt_idx_v.at[pl.ds(g * CHUNK + h * HALF, HALF)]],
                tgt_rows.at[par, pl.ds(h * HALF, HALF)], sems.at[par]))
            for c in range(NUM_CTX):
                cps.append(pltpu.async_copy(
                    w_ctx_hbm.at[ctx_idx_v.at[c, pl.ds(g * CHUNK + h * HALF,
                                                       HALF)]],
                    ctx_rows.at[par, pl.ds(c * CHUNK + h * HALF, HALF)],
                    sems.at[par]))
        return cps

    def compute(g, par):
        @pl.loop(0, CHUNK)
        def _(b):
            t_chunks = [tgt_rows[par, b, pl.ds(k * LANES, LANES)]
                        for k in range(K_CHUNKS)]
            accs = []
            for c in range(NUM_CTX):
                prods = [t_chunks[k] * ctx_rows[par, c * CHUNK + b,
                                                pl.ds(k * LANES, LANES)]
                         for k in range(K_CHUNKS)]
                while len(prods) > 1:
                    prods = [prods[i] + prods[i + 1]
                             for i in range(0, len(prods) - 1, 2)] + (
                                 [prods[-1]] if len(prods) % 2 else [])
                accs.append(prods[0])
            sums = [plsc.cumsum(a) for a in accs]
            idx_b = jnp.full((LANES,), (g * CHUNK + b) * NUM_CTX, jnp.int32)
            for c in range(NUM_CTX):
                plsc.store_scatter(out_v, [idx_b + c], sums[c], mask=last_lane)

    cps = issue_gathers(0, 0)
    for g in range(NUM_CHUNKS):
        par = g % 2
        for cp in cps:
            cp.wait()
        if g + 1 < NUM_CHUNKS:
            cps = issue_gathers(g + 1, 1 - par)
        compute(g, par)

    pltpu.sync_copy(out_v,
                    out_hbm.at[pl.ds(base * NUM_CTX, B_PER_W * NUM_CTX)])


def kernel(target, context, W_target, W_context):
    tgt_idx = target.reshape(BATCH).astype(jnp.int32)
    ctx_idx = context.T.astype(jnp.int32)

    mesh = plsc.VectorSubcoreMesh(core_axis_name="core",
                                  subcore_axis_name="subcore")
    sc_call = pl.kernel(
        _sc_kernel_body,
        out_type=jax.ShapeDtypeStruct((BATCH * NUM_CTX,), jnp.float32),
        mesh=mesh,
        scratch_types=[
            pltpu.VMEM((B_PER_W,), jnp.int32),
            pltpu.VMEM((NUM_CTX, B_PER_W), jnp.int32),
            pltpu.VMEM((2, CHUNK, EMBED), jnp.float32),
            pltpu.VMEM((2, NUM_CTX * CHUNK, EMBED), jnp.float32),
            pltpu.VMEM((B_PER_W * NUM_CTX,), jnp.float32),
            pltpu.SemaphoreType.DMA((2,)),
        ],
        compiler_params=pltpu.CompilerParams(needs_layout_passes=False),
    )
    return sc_call(tgt_idx, ctx_idx, W_target, W_context).reshape(BATCH, NUM_CTX)


# speedup vs baseline: 1.0249x; 1.0249x over previous
import jax
import jax.numpy as jnp
from jax import lax
from jax.experimental import pallas as pl
from jax.experimental.pallas import tpu as pltpu
from jax.experimental.pallas import tpu_sc as plsc

VOCAB = 100000
EMBED = 128
BATCH = 4096
NUM_CTX = 5

NUM_CORES = 2
NUM_SUBCORES = 16
NUM_WORKERS = NUM_CORES * NUM_SUBCORES
B_PER_W = BATCH // NUM_WORKERS
CHUNK = 64
NUM_CHUNKS = B_PER_W // CHUNK
LANES = 16
K_CHUNKS = EMBED // LANES


def _sc_kernel_body(tgt_idx_hbm, ctx_idx_hbm, w_tgt_hbm, w_ctx_hbm, out_hbm,
                    tgt_idx_v, ctx_idx_v, tgt_rows, ctx_rows, out_v, sems):
    wid = lax.axis_index("subcore") * NUM_CORES + lax.axis_index("core")
    base = wid * B_PER_W

    icp1 = pltpu.async_copy(tgt_idx_hbm.at[pl.ds(base, B_PER_W)], tgt_idx_v,
                            sems.at[2])
    icp2 = pltpu.async_copy(ctx_idx_hbm.at[:, pl.ds(base, B_PER_W)], ctx_idx_v,
                            sems.at[2])
    icp1.wait()
    icp2.wait()

    last_lane = lax.iota(jnp.int32, LANES) == (LANES - 1)

    def issue_gathers(g, par):
        cps = [pltpu.async_copy(
            w_tgt_hbm.at[tgt_idx_v.at[pl.ds(g * CHUNK, CHUNK)]],
            tgt_rows.at[par], sems.at[par])]
        for c in range(NUM_CTX):
            cps.append(pltpu.async_copy(
                w_ctx_hbm.at[ctx_idx_v.at[c, pl.ds(g * CHUNK, CHUNK)]],
                ctx_rows.at[par, pl.ds(c * CHUNK, CHUNK)], sems.at[par]))
        return cps

    def compute(g, par):
        @pl.loop(0, CHUNK)
        def _(b):
            t_chunks = [tgt_rows[par, b, pl.ds(k * LANES, LANES)]
                        for k in range(K_CHUNKS)]
            accs = []
            for c in range(NUM_CTX):
                prods = [t_chunks[k] * ctx_rows[par, c * CHUNK + b,
                                                pl.ds(k * LANES, LANES)]
                         for k in range(K_CHUNKS)]
                while len(prods) > 1:
                    prods = [prods[i] + prods[i + 1]
                             for i in range(0, len(prods) - 1, 2)] + (
                                 [prods[-1]] if len(prods) % 2 else [])
                accs.append(prods[0])
            sums = [plsc.cumsum(a) for a in accs]
            idx_b = jnp.full((LANES,), (g * CHUNK + b) * NUM_CTX, jnp.int32)
            for c in range(NUM_CTX):
                plsc.store_scatter(out_v, [idx_b + c], sums[c], mask=last_lane)

    ocps = []
    cps = issue_gathers(0, 0)
    for g in range(NUM_CHUNKS):
        par = g % 2
        for cp in cps:
            cp.wait()
        if g + 1 < NUM_CHUNKS:
            cps = issue_gathers(g + 1, 1 - par)
        compute(g, par)
        ocps.append(pltpu.async_copy(
            out_v.at[pl.ds(g * CHUNK * NUM_CTX, CHUNK * NUM_CTX)],
            out_hbm.at[pl.ds(base * NUM_CTX + g * CHUNK * NUM_CTX,
                             CHUNK * NUM_CTX)], sems.at[2]))

    for ocp in ocps:
        ocp.wait()


def kernel(target, context, W_target, W_context):
    tgt_idx = target.reshape(BATCH).astype(jnp.int32)
    ctx_idx = context.T.astype(jnp.int32)

    mesh = plsc.VectorSubcoreMesh(core_axis_name="core",
                                  subcore_axis_name="subcore")
    sc_call = pl.kernel(
        _sc_kernel_body,
        out_type=jax.ShapeDtypeStruct((BATCH * NUM_CTX,), jnp.float32),
        mesh=mesh,
        scratch_types=[
            pltpu.VMEM((B_PER_W,), jnp.int32),
            pltpu.VMEM((NUM_CTX, B_PER_W), jnp.int32),
            pltpu.VMEM((2, CHUNK, EMBED), jnp.float32),
            pltpu.VMEM((2, NUM_CTX * CHUNK, EMBED), jnp.float32),
            pltpu.VMEM((B_PER_W * NUM_CTX,), jnp.float32),
            pltpu.SemaphoreType.DMA((3,)),
        ],
        compiler_params=pltpu.CompilerParams(needs_layout_passes=False),
    )
    return sc_call(tgt_idx, ctx_idx, W_target, W_context).reshape(BATCH, NUM_CTX)
